# Initial kernel scaffold; baseline (speedup 1.0000x reference)
import jax
import jax.numpy as jnp
from jax.experimental import pallas as pl


def kernel(x0, x1, x2, node_idx0, node_idx1, node_idx2, hedge_idx0, hedge_idx1, hedge_idx2, W1, b1, ae1, W2, b2, ae2, Wa, ba, Wb, bb, Wc, bc, Wo, bo, g1, be1, g2, be2, Wf, bf):
    raise NotImplementedError("write your pallas kernel here")


# speedup vs baseline: 9.4406x; 9.4406x over previous
import functools
import math

import jax
import jax.numpy as jnp
from jax.experimental import pallas as pl
from jax.experimental.pallas import tpu as pltpu

N_NODES = 10000
N_HE = 2000
P = 160000
E_PAD = 2048
EB = 16
VB = 1000
NV = N_NODES // VB
NEG = -1e30


def _build_incidence(nid, eid):
    cb = jnp.zeros((EB, N_NODES, 128), jnp.float32)
    return cb.at[eid >> 7, nid, eid & 127].add(1.0)


def _v2e_body(cb_ref, x_ref, ae_ref, y_ref, al_ref, acc, dacc):
    v = pl.program_id(1)

    @pl.when(v == 0)
    def _():
        acc[...] = jnp.zeros_like(acc)
        dacc[...] = jnp.zeros_like(dacc)

    cb = cb_ref[0]
    x = x_ref[...]
    acc[...] += jax.lax.dot_general(cb, x, (((0,), (0,)), ((), ())),
                                    preferred_element_type=jnp.float32)
    ones = jnp.ones((cb.shape[0], 1), jnp.float32)
    dacc[...] += jax.lax.dot_general(cb, ones, (((0,), (0,)), ((), ())),
                                     preferred_element_type=jnp.float32)

    @pl.when(v == NV - 1)
    def _():
        deg = jnp.maximum(dacc[...], 1.0)
        yb = acc[...] / deg
        y_ref[...] = yb
        a = jax.lax.dot_general(yb, ae_ref[...], (((1,), (1,)), ((), ())),
                                preferred_element_type=jnp.float32)
        al_ref[...] = jnp.where(a >= 0, a, 0.2 * a)


def _v2e(cb, xp, ae):
    cw = xp.shape[1]
    return pl.pallas_call(
        _v2e_body,
        grid=(EB, NV),
        in_specs=[
            pl.BlockSpec((1, VB, 128), lambda e, v: (e, v, 0)),
            pl.BlockSpec((VB, cw), lambda e, v: (v, 0)),
            pl.BlockSpec((1, cw), lambda e, v: (0, 0)),
        ],
        out_specs=[
            pl.BlockSpec((128, cw), lambda e, v: (e, 0)),
            pl.BlockSpec((128, 1), lambda e, v: (e, 0)),
        ],
        out_shape=[
            jax.ShapeDtypeStruct((E_PAD, cw), jnp.float32),
            jax.ShapeDtypeStruct((E_PAD, 1), jnp.float32),
        ],
        scratch_shapes=[
            pltpu.VMEM((128, cw), jnp.float32),
            pltpu.VMEM((128, 1), jnp.float32),
        ],
    )(cb, xp.reshape(N_NODES, cw), ae.reshape(1, cw))


def _e2v_body(fuse, elu, cb_ref, y_ref, af_ref, ab_ref, w_ref, b_ref,
              out_ref, accv, accz):
    e = pl.program_id(1)

    @pl.when(e == 0)
    def _():
        accv[...] = jnp.zeros_like(accv)
        accz[...] = jnp.zeros_like(accz)

    gm = jnp.max(af_ref[...])
    ub = jnp.exp(ab_ref[...] - gm)
    cb = cb_ref[0]
    accv[...] += jnp.dot(cb, ub * y_ref[...],
                         preferred_element_type=jnp.float32)
    accz[...] += jnp.dot(cb, ub, preferred_element_type=jnp.float32)

    @pl.when(e == EB - 1)
    def _():
        h = accv[...] / jnp.maximum(accz[...], 1e-12)
        if elu:
            h = jnp.where(h > 0, h, jnp.exp(h) - 1.0)
        if fuse:
            out_ref[...] = jnp.dot(h, w_ref[...],
                                   preferred_element_type=jnp.float32) + b_ref[...]
        else:
            out_ref[...] = h


def _e2v(cb, y, alpha, w=None, b=None, elu=False):
    cw = y.shape[1]
    fuse = w is not None
    cout = w.shape[1] if fuse else cw
    if not fuse:
        w = jnp.zeros((cw, 8), jnp.float32)
        b = jnp.zeros((8,), jnp.float32)
    b = b.reshape(1, -1)
    return pl.pallas_call(
        functools.partial(_e2v_body, fuse, elu),
        grid=(NV, EB),
        in_specs=[
            pl.BlockSpec((1, VB, 128), lambda v, e: (e, v, 0)),
            pl.BlockSpec((128, cw), lambda v, e: (e, 0)),
            pl.BlockSpec((E_PAD, 1), lambda v, e: (0, 0)),
            pl.BlockSpec((128, 1), lambda v, e: (e, 0)),
            pl.BlockSpec(w.shape, lambda v, e: (0, 0)),
            pl.BlockSpec(b.shape, lambda v, e: (0, 0)),
        ],
        out_specs=pl.BlockSpec((VB, cout), lambda v, e: (v, 0)),
        out_shape=jax.ShapeDtypeStruct((N_NODES, cout), jnp.float32),
        scratch_shapes=[
            pltpu.VMEM((VB, cw), jnp.float32),
            pltpu.VMEM((VB, 1), jnp.float32),
        ],
    )(cb, y, alpha, alpha, w, b)


def _proj_body(x_ref, w_ref, b_ref, o_ref):
    o_ref[...] = jnp.dot(x_ref[...], w_ref[...],
                         preferred_element_type=jnp.float32) + b_ref[...]


def _proj(x, w, b):
    cin, cout = w.shape
    return pl.pallas_call(
        _proj_body,
        grid=(NV,),
        in_specs=[
            pl.BlockSpec((VB, cin), lambda v: (v, 0)),
            pl.BlockSpec((cin, cout), lambda v: (0, 0)),
            pl.BlockSpec((1, cout), lambda v: (0, 0)),
        ],
        out_specs=pl.BlockSpec((VB, cout), lambda v: (v, 0)),
        out_shape=jax.ShapeDtypeStruct((N_NODES, cout), jnp.float32),
    )(x, w, b.reshape(1, cout))


def _attn_body(n, nb, x_ref, wa_ref, ba_ref, wb_ref, bb_ref, wc_ref, bc_ref,
               wo_ref, bo_ref, g_ref, be_ref, o_ref, num, den, m):
    i = pl.program_id(0)

    @pl.when(i == 0)
    def _():
        num[...] = jnp.zeros_like(num)
        den[...] = jnp.zeros_like(den)
        m[...] = jnp.full_like(m, NEG)

    xb = x_ref[...]
    rows = i * xb.shape[0] + jax.lax.broadcasted_iota(jnp.int32, (xb.shape[0], 1), 0)
    valid = rows < n
    xb = jnp.where(valid, xb, 0.0)
    a = jnp.tanh(jnp.dot(xb, wa_ref[...], preferred_element_type=jnp.float32)
                 + ba_ref[...])
    g = jax.nn.sigmoid(jnp.dot(xb, wb_ref[...], preferred_element_type=jnp.float32)
                       + bb_ref[...])
    s = jnp.dot(a * g, wc_ref[...], preferred_element_type=jnp.float32) + bc_ref[...]
    s = jnp.where(valid, s, NEG)
    mo = jnp.max(m[...])
    mn = jnp.maximum(mo, jnp.max(s))
    scale = jnp.exp(mo - mn)
    wgt = jnp.exp(s - mn)
    num[...] = num[...] * scale + jax.lax.dot_general(
        wgt, xb, (((0,), (0,)), ((), ())), preferred_element_type=jnp.float32)
    den[...] = den[...] * scale + jnp.sum(wgt)
    m[...] = jnp.full_like(m, mn)

    @pl.when(i == nb - 1)
    def _():
        gf = num[...] / den[...]
        o = jax.lax.dot_general(gf, wo_ref[...], (((1,), (1,)), ((), ())),
                                preferred_element_type=jnp.float32) + bo_ref[...]
        mu = jnp.mean(o)
        var = jnp.mean((o - mu) ** 2)
        o_ref[...] = (o - mu) * jax.lax.rsqrt(var + 1e-5) * g_ref[...] + be_ref[...]


def _attn_pool(x, n, wa, ba, wb, bb, wc, bc, wo, bo, g1, be1):
    rows = x.shape[0]
    vb = min(VB, rows)
    nb = rows // vb
    ah = wa.shape[1]
    return pl.pallas_call(
        functools.partial(_attn_body, n, nb),
        grid=(nb,),
        in_specs=[
            pl.BlockSpec((vb, 128), lambda i: (i, 0)),
            pl.BlockSpec((128, ah), lambda i: (0, 0)),
            pl.BlockSpec((1, ah), lambda i: (0, 0)),
            pl.BlockSpec((128, ah), lambda i: (0, 0)),
            pl.BlockSpec((1, ah), lambda i: (0, 0)),
            pl.BlockSpec((ah, 1), lambda i: (0, 0)),
            pl.BlockSpec((1, 1), lambda i: (0, 0)),
            pl.BlockSpec((128, 128), lambda i: (0, 0)),
            pl.BlockSpec((1, 128), lambda i: (0, 0)),
            pl.BlockSpec((1, 128), lambda i: (0, 0)),
            pl.BlockSpec((1, 128), lambda i: (0, 0)),
        ],
        out_specs=pl.BlockSpec((1, 128), lambda i: (0, 0)),
        out_shape=jax.ShapeDtypeStruct((1, 128), jnp.float32),
        scratch_shapes=[
            pltpu.VMEM((1, 128), jnp.float32),
            pltpu.VMEM((1, 128), jnp.float32),
            pltpu.VMEM((1, 128), jnp.float32),
        ],
    )(x, wa, ba.reshape(1, ah), wb, bb.reshape(1, ah), wc.reshape(ah, 1),
      bc.reshape(1, 1), wo, bo.reshape(1, 128), g1.reshape(1, 128),
      be1.reshape(1, 128))


def _head_body(z_ref, g_ref, be_ref, wf_ref, bf_ref, o_ref):
    z = z_ref[...]
    mu = jnp.mean(z)
    var = jnp.mean((z - mu) ** 2)
    z = (z - mu) * jax.lax.rsqrt(var + 1e-5) * g_ref[...] + be_ref[...]
    o_ref[...] = jnp.dot(z, wf_ref[...], preferred_element_type=jnp.float32) \
        + bf_ref[...]


def _head(parts, g2, be2, wf, bf):
    z = jnp.concatenate(parts, axis=1)
    d = z.shape[1]
    ncls = wf.shape[1]
    return pl.pallas_call(
        _head_body,
        in_specs=[
            pl.BlockSpec((1, d), lambda: (0, 0)),
            pl.BlockSpec((1, d), lambda: (0, 0)),
            pl.BlockSpec((1, d), lambda: (0, 0)),
            pl.BlockSpec((d, ncls), lambda: (0, 0)),
            pl.BlockSpec((1, ncls), lambda: (0, 0)),
        ],
        out_specs=pl.BlockSpec((1, ncls), lambda: (0, 0)),
        out_shape=jax.ShapeDtypeStruct((1, ncls), jnp.float32),
    )(z, g2.reshape(1, d), be2.reshape(1, d), wf, bf.reshape(1, ncls))


def kernel(x0, x1, x2, node_idx0, node_idx1, node_idx2, hedge_idx0,
           hedge_idx1, hedge_idx2, W1, b1, ae1, W2, b2, ae2, Wa, ba, Wb, bb,
           Wc, bc, Wo, bo, g1, be1, g2, be2, Wf, bf):
    xouts, youts = [], []
    for x, nid, eid in zip((x0, x1, x2), (node_idx0, node_idx1, node_idx2),
                           (hedge_idx0, hedge_idx1, hedge_idx2)):
        cb = _build_incidence(nid, eid)
        xp = _proj(x, W1, b1)
        y1, a1 = _v2e(cb, xp, ae1)
        x2p = _e2v(cb, y1, a1, w=W2, b=b2, elu=True)
        y2, a2 = _v2e(cb, x2p, ae2)
        h2 = _e2v(cb, y2, a2)
        yv, _ = _v2e(cb, h2, ae2)
        xg = _attn_pool(h2, N_NODES, Wa, ba, Wb, bb, Wc, bc, Wo, bo, g1, be1)
        yg = _attn_pool(yv, N_HE, Wa, ba, Wb, bb, Wc, bc, Wo, bo, g1, be1)
        xouts.append(xg)
        youts.append(yg)
    return _head(xouts + youts, g2, be2, Wf, bf)
